# baseline (speedup 1.0000x reference)
import functools

import jax
import jax.numpy as jnp
from jax import lax
from jax.experimental import pallas as pl
from jax.experimental.pallas import tpu as pltpu
from jax.experimental.pallas import tpu_sc as plsc

N_FACTORS = 32
BATCH = 16384

_info = plsc.get_sparse_core_info()
NC, NS, L = _info.num_cores, _info.num_subcores, _info.num_lanes
NW = NC * NS
BPW = BATCH // NW
CHUNK = 128
N_CHUNKS = BPW // CHUNK


def _mf_body(uf_hbm, if_hbm, users_hbm, items_hbm, out_hbm,
             uidx_v, iidx_v, urows_v, irows_v, out_v, sem):
    wid = lax.axis_index("s") * NC + lax.axis_index("c")
    base = wid * BPW

    pltpu.sync_copy(users_hbm.at[pl.ds(base, BPW)], uidx_v)
    pltpu.sync_copy(items_hbm.at[pl.ds(base, BPW)], iidx_v)

    copies = []
    for k in range(N_CHUNKS):
        sl = pl.ds(k * CHUNK, CHUNK)
        copies.append(pltpu.async_copy(uf_hbm.at[uidx_v.at[sl]],
                                       urows_v.at[sl],
                                       sem))
        copies.append(pltpu.async_copy(if_hbm.at[iidx_v.at[sl]],
                                       irows_v.at[sl],
                                       sem))
    for c in copies:
        c.wait()

    lane = lax.iota(jnp.int32, L)

    def group_body(g, _):
        rows = g * L + lane
        acc = jnp.zeros((L,), jnp.float32)
        for d in range(N_FACTORS):
            col = jnp.full((L,), d, jnp.int32)
            uu = plsc.load_gather(urows_v, [rows, col])
            vv = plsc.load_gather(irows_v, [rows, col])
            acc = acc + uu * vv
        out_v[pl.ds(g * L, L)] = acc
        return 0

    lax.fori_loop(0, BPW // L, group_body, 0)

    pltpu.sync_copy(out_v, out_hbm.at[pl.ds(base, BPW)])


@functools.partial(jax.jit, static_argnames=())
def kernel(user_factors, item_factors, users, items):
    users = users.astype(jnp.int32)
    items = items.astype(jnp.int32)
    mesh = plsc.VectorSubcoreMesh(core_axis_name="c", subcore_axis_name="s")
    run = pl.kernel(
        _mf_body,
        out_type=jax.ShapeDtypeStruct((BATCH,), jnp.float32),
        mesh=mesh,
        scratch_types=[
            pltpu.VMEM((BPW,), jnp.int32),
            pltpu.VMEM((BPW,), jnp.int32),
            pltpu.VMEM((BPW, N_FACTORS), jnp.float32),
            pltpu.VMEM((BPW, N_FACTORS), jnp.float32),
            pltpu.VMEM((BPW,), jnp.float32),
            pltpu.SemaphoreType.DMA,
        ],
        compiler_params=pltpu.CompilerParams(
            needs_layout_passes=False, use_tc_tiling_on_sc=False),
    )
    return run(user_factors, item_factors, users, items)
